# baseline (speedup 1.0000x reference)
import jax
import jax.numpy as jnp
from jax.experimental import pallas as pl
from jax.experimental.pallas import tpu as pltpu

_NCHUNK = 8


def kernel(memory, input_logits):
    size, dim = memory.shape
    num = input_logits.shape[0]
    bulk = size - num
    chunk = pl.cdiv(bulk, _NCHUNK)

    def body(mem_ref, logits_ref, out_ref, idx_ref, sem_bulk, sem_log):
        copies = []
        for c in range(_NCHUNK):
            lo = num + c * chunk
            n = min(chunk, size - (num + c * chunk))
            if n <= 0:
                continue
            copies.append(
                pltpu.make_async_copy(
                    mem_ref.at[pl.ds(lo, n)],
                    out_ref.at[pl.ds(lo, n)],
                    sem_bulk.at[c],
                )
            )
        copies.append(
            pltpu.make_async_copy(
                logits_ref, out_ref.at[pl.ds(0, num)], sem_log
            )
        )
        for cp in copies:
            cp.start()
        idx_ref[0] = jnp.int32(num % size)
        for cp in copies:
            cp.wait()

    memory_new, new_index = pl.pallas_call(
        body,
        in_specs=[
            pl.BlockSpec(memory_space=pl.ANY),
            pl.BlockSpec(memory_space=pl.ANY),
        ],
        out_specs=[
            pl.BlockSpec(memory_space=pl.ANY),
            pl.BlockSpec(memory_space=pltpu.SMEM),
        ],
        out_shape=[
            jax.ShapeDtypeStruct((size, dim), memory.dtype),
            jax.ShapeDtypeStruct((1,), jnp.int32),
        ],
        scratch_shapes=[
            pltpu.SemaphoreType.DMA((_NCHUNK,)),
            pltpu.SemaphoreType.DMA,
        ],
    )(memory, input_logits)
    return (memory_new, new_index[0])


# speedup vs baseline: 14.7531x; 14.7531x over previous
import jax
import jax.numpy as jnp
from jax.experimental import pallas as pl
from jax.experimental.pallas import tpu as pltpu

_W = 128
_BLOCK = 8192


def kernel(memory, input_logits):
    size, dim = memory.shape
    num = input_logits.shape[0]
    assert (size * dim) % _W == 0 and (num * dim) % _W == 0
    rows = size * dim // _W
    lrows = num * dim // _W
    assert lrows <= _BLOCK
    mem2 = memory.reshape(rows, _W)
    log2 = input_logits.reshape(lrows, _W)
    grid = (pl.cdiv(rows, _BLOCK),)

    def body(mem_ref, logits_ref, out_ref, idx_ref):
        i = pl.program_id(0)

        @pl.when(i == 0)
        def _():
            out_ref[0:lrows, :] = logits_ref[...]
            out_ref[lrows:_BLOCK, :] = mem_ref[lrows:_BLOCK, :]
            idx_ref[0] = jnp.int32(num % size)

        @pl.when(i > 0)
        def _():
            out_ref[...] = mem_ref[...]

    out2, new_index = pl.pallas_call(
        body,
        grid=grid,
        in_specs=[
            pl.BlockSpec((_BLOCK, _W), lambda i: (i, 0)),
            pl.BlockSpec((lrows, _W), lambda i: (0, 0)),
        ],
        out_specs=[
            pl.BlockSpec((_BLOCK, _W), lambda i: (i, 0)),
            pl.BlockSpec(memory_space=pltpu.SMEM),
        ],
        out_shape=[
            jax.ShapeDtypeStruct((rows, _W), memory.dtype),
            jax.ShapeDtypeStruct((1,), jnp.int32),
        ],
    )(mem2, log2)
    return (out2.reshape(size, dim), new_index[0])
